# baseline (speedup 1.0000x reference)
import jax
import jax.numpy as jnp
import numpy as np
from jax.experimental import pallas as pl
from jax.experimental.pallas import tpu as pltpu

V = 128
D = 64
DD = D * D
DERTA = 10.0

_Pa = np.zeros((D, DD), dtype=np.float32)
_Pb = np.zeros((D, DD), dtype=np.float32)
for _a in range(D):
    _Pa[_a, _a * D:(_a + 1) * D] = 1.0
for _b in range(D):
    _Pb[_b, _b::D] = 1.0

BI = 4


def _prep_kernel(cos_ref, feat_ref, pa_ref, pb_ref,
                 eij_ref, sij_ref, a_ref, b_ref):
    eij = 0.5 * (1.0 - cos_ref[...])
    eij_ref[...] = eij
    sij_ref[...] = jnp.exp(-DERTA * eij)
    feats = feat_ref[...]
    a_ref[...] = jnp.dot(feats, pa_ref[...], preferred_element_type=jnp.float32)
    b_ref[...] = jnp.dot(feats, pb_ref[...], preferred_element_type=jnp.float32)


NBUF = 4
NSTEPS = V // BI


def _stream_kernel(a_ref, b_ref, c_hbm, scratch, sems):
    i = pl.program_id(0)
    s = jax.lax.rem(i, NBUF)

    @pl.when(i >= NBUF)
    def _():
        pltpu.make_async_copy(
            scratch.at[s],
            c_hbm.at[pl.ds((i - NBUF) * BI, BI)],
            sems.at[s],
        ).wait()

    scratch[s] = jnp.zeros((BI, 512, 8, 128), jnp.float32)
    pltpu.make_async_copy(
        scratch.at[s],
        c_hbm.at[pl.ds(i * BI, BI)],
        sems.at[s],
    ).start()

    @pl.when(i == NSTEPS - 1)
    def _():
        for dj in range(NBUF):
            j = NSTEPS - NBUF + dj
            pltpu.make_async_copy(
                scratch.at[j % NBUF],
                c_hbm.at[pl.ds(j * BI, BI)],
                sems.at[j % NBUF],
            ).wait()


@jax.jit
def kernel(costheta, features):
    eij, sij, a_full, b_full = pl.pallas_call(
        _prep_kernel,
        out_shape=[
            jax.ShapeDtypeStruct((V, V), jnp.float32),
            jax.ShapeDtypeStruct((V, V), jnp.float32),
            jax.ShapeDtypeStruct((V, DD), jnp.float32),
            jax.ShapeDtypeStruct((V, DD), jnp.float32),
        ],
    )(costheta, features, _Pa, _Pb)

    c_flat = pl.pallas_call(
        _stream_kernel,
        grid=(V // BI,),
        in_specs=[
            pl.BlockSpec((1, BI, DD), lambda i: (i, 0, 0)),
            pl.BlockSpec((V, DD), lambda i: (0, 0)),
        ],
        out_specs=pl.BlockSpec(memory_space=pl.ANY),
        out_shape=jax.ShapeDtypeStruct((V, 512, 8, 128), jnp.float32),
        scratch_shapes=[
            pltpu.VMEM((NBUF, BI, 512, 8, 128), jnp.float32),
            pltpu.SemaphoreType.DMA((NBUF,)),
        ],
    )(a_full.reshape(V // BI, BI, DD), b_full)
    return (eij, sij, c_flat.reshape(V, V, D, D))


# speedup vs baseline: 1.1847x; 1.1847x over previous
import functools

import jax
import jax.numpy as jnp
import numpy as np
from jax import lax
from jax.experimental import pallas as pl
from jax.experimental.pallas import tpu as pltpu
from jax.experimental.pallas import tpu_sc as plsc

V = 128
D = 64
DD = D * D
DERTA = 10.0
L = 16

SPLIT = 64

NC = 2
NS = 16
NW = NC * NS
IPR = (V - SPLIT) // NW
K = 2
NCHUNK = V // K
UNROLL = 16
NPAIR = NCHUNK // 2

BI = 4
NBUF = 2
NSTEPS = SPLIT // BI

_Pa = np.zeros((D, DD), dtype=np.float32)
_Pb = np.zeros((D, DD), dtype=np.float32)
for _a in range(D):
    _Pa[_a, _a * D:(_a + 1) * D] = 1.0
for _b in range(D):
    _Pb[_b, _b::D] = 1.0


def _prep_kernel(cos_ref, feat_ref, pa_ref, pb_ref,
                 eij_ref, sij_ref, a_ref, b_ref):
    eij = 0.5 * (1.0 - cos_ref[...])
    eij_ref[...] = eij
    sij_ref[...] = jnp.exp(-DERTA * eij)
    feats = feat_ref[...]
    a_ref[...] = jnp.dot(feats, pa_ref[...], preferred_element_type=jnp.float32)
    b_ref[...] = jnp.dot(feats, pb_ref[...], preferred_element_type=jnp.float32)


def _stream_tc(a_ref, b_ref, c_hbm, scratch, sems):
    i = pl.program_id(0)
    s = jax.lax.rem(i, NBUF)

    @pl.when(i >= NBUF)
    def _():
        pltpu.make_async_copy(
            scratch.at[s],
            c_hbm.at[pl.ds((i - NBUF) * BI, BI)],
            sems.at[s],
        ).wait()

    scratch[s] = a_ref[0][:, None, :] * b_ref[...][None, :, :]
    pltpu.make_async_copy(
        scratch.at[s],
        c_hbm.at[pl.ds(i * BI, BI)],
        sems.at[s],
    ).start()

    @pl.when(i == NSTEPS - 1)
    def _():
        for dj in range(NBUF):
            j = NSTEPS - NBUF + dj
            pltpu.make_async_copy(
                scratch.at[j % NBUF],
                c_hbm.at[pl.ds(j * BI, BI)],
                sems.at[j % NBUF],
            ).wait()


def _stream_sc(a_hbm, b_hbm, out_hbm, a_v, b_v, o_v, sem_a, sem_b, sem_o):
    wid = lax.axis_index("s") * NC + lax.axis_index("c")
    i0 = wid * IPR

    pltpu.async_copy(a_hbm.at[pl.ds(i0, IPR)], a_v, sem_a).wait()

    pltpu.async_copy(b_hbm.at[pl.ds(0, K)], b_v.at[0], sem_b)
    pltpu.async_copy(b_hbm.at[pl.ds(K, K)], b_v.at[1], sem_b)

    def pair(q, _):
        for sub in range(2):
            buf = sub
            c = 2 * q + sub
            pltpu.make_async_copy(
                b_hbm.at[pl.ds(c * K, K)], b_v.at[buf], sem_b).wait()

            @pl.when(q >= 1)
            def _():
                for i_loc in range(IPR):
                    pltpu.make_async_copy(
                        o_v.at[buf, i_loc],
                        out_hbm.at[i0 + i_loc, pl.ds((c - 2) * K, K)],
                        sem_o).wait()

            def body(k, _, buf=buf):
                for u in range(UNROLL):
                    sl = pl.ds((k * UNROLL + u) * L, L)
                    avs = [a_v[i_loc, sl] for i_loc in range(IPR)]
                    bvs = [b_v[buf, jj, sl] for jj in range(K)]
                    for i_loc in range(IPR):
                        for jj in range(K):
                            o_v[buf, i_loc, jj, sl] = avs[i_loc] * bvs[jj]
                return 0

            lax.fori_loop(0, DD // L // UNROLL, body, 0)

            @pl.when(q <= NPAIR - 2)
            def _():
                pltpu.async_copy(
                    b_hbm.at[pl.ds((c + 2) * K, K)], b_v.at[buf], sem_b)

            for i_loc in range(IPR):
                pltpu.async_copy(
                    o_v.at[buf, i_loc],
                    out_hbm.at[i0 + i_loc, pl.ds(c * K, K)],
                    sem_o)
        return 0

    lax.fori_loop(0, NPAIR, pair, 0)

    for sub in range(2):
        c = NCHUNK - 2 + sub
        for i_loc in range(IPR):
            pltpu.make_async_copy(
                o_v.at[sub, i_loc],
                out_hbm.at[i0 + i_loc, pl.ds(c * K, K)],
                sem_o).wait()


_sc_call = functools.partial(
    pl.kernel,
    out_type=jax.ShapeDtypeStruct((V - SPLIT, V, DD), jnp.float32),
    mesh=plsc.VectorSubcoreMesh(core_axis_name="c", subcore_axis_name="s"),
    scratch_types=[
        pltpu.VMEM((IPR, DD), jnp.float32),
        pltpu.VMEM((2, K, DD), jnp.float32),
        pltpu.VMEM((2, IPR, K, DD), jnp.float32),
        pltpu.SemaphoreType.DMA,
        pltpu.SemaphoreType.DMA,
        pltpu.SemaphoreType.DMA,
    ],
)(_stream_sc)


@jax.jit
def kernel(costheta, features):
    eij, sij, a_full, b_full = pl.pallas_call(
        _prep_kernel,
        out_shape=[
            jax.ShapeDtypeStruct((V, V), jnp.float32),
            jax.ShapeDtypeStruct((V, V), jnp.float32),
            jax.ShapeDtypeStruct((V, DD), jnp.float32),
            jax.ShapeDtypeStruct((V, DD), jnp.float32),
        ],
    )(costheta, features, _Pa, _Pb)

    c_bot = _sc_call(lax.slice(a_full, (SPLIT, 0), (V, DD)), b_full)

    c_top = pl.pallas_call(
        _stream_tc,
        grid=(NSTEPS,),
        in_specs=[
            pl.BlockSpec((1, BI, DD), lambda i: (i, 0, 0)),
            pl.BlockSpec((V, DD), lambda i: (0, 0)),
        ],
        out_specs=pl.BlockSpec(memory_space=pl.ANY),
        out_shape=jax.ShapeDtypeStruct((V, V, DD), jnp.float32),
        scratch_shapes=[
            pltpu.VMEM((NBUF, BI, V, DD), jnp.float32),
            pltpu.SemaphoreType.DMA((NBUF,)),
        ],
    )(a_full[:SPLIT].reshape(SPLIT // BI, BI, DD), b_full)

    c_flat = lax.dynamic_update_slice(c_top, c_bot, (SPLIT, 0, 0))
    return (eij, sij, c_flat.reshape(V, V, D, D))


# speedup vs baseline: 1.6223x; 1.3694x over previous
import jax
import jax.numpy as jnp
import numpy as np
from jax.experimental import pallas as pl
from jax.experimental.pallas import tpu as pltpu

V = 128
D = 64
DD = D * D
DERTA = 10.0

_Pa = np.zeros((D, DD), dtype=np.float32)
_Pb = np.zeros((D, DD), dtype=np.float32)
for _a in range(D):
    _Pa[_a, _a * D:(_a + 1) * D] = 1.0
for _b in range(D):
    _Pb[_b, _b::D] = 1.0

BI = 8


def _prep_kernel(cos_ref, feat_ref, pa_ref, pb_ref,
                 eij_ref, sij_ref, a_ref, b_ref):
    eij = 0.5 * (1.0 - cos_ref[...])
    eij_ref[...] = eij
    sij_ref[...] = jnp.exp(-DERTA * eij)
    feats = feat_ref[...]
    a_ref[...] = jnp.dot(feats, pa_ref[...], preferred_element_type=jnp.float32)
    b_ref[...] = jnp.dot(feats, pb_ref[...], preferred_element_type=jnp.float32)


NBUF = 2
NSTEPS = V // BI


def _stream_kernel(a_ref, b_ref, c_hbm, scratch, sems):
    i = pl.program_id(0)
    s = jax.lax.rem(i, NBUF)

    @pl.when(i >= NBUF)
    def _():
        pltpu.make_async_copy(
            scratch.at[s],
            c_hbm.at[pl.ds((i - NBUF) * BI, BI)],
            sems.at[s],
        ).wait()

    scratch[s] = a_ref[0][:, None, :] * b_ref[...][None, :, :]
    pltpu.make_async_copy(
        scratch.at[s],
        c_hbm.at[pl.ds(i * BI, BI)],
        sems.at[s],
    ).start()

    @pl.when(i == NSTEPS - 1)
    def _():
        for dj in range(NBUF):
            j = NSTEPS - NBUF + dj
            pltpu.make_async_copy(
                scratch.at[j % NBUF],
                c_hbm.at[pl.ds(j * BI, BI)],
                sems.at[j % NBUF],
            ).wait()


@jax.jit
def kernel(costheta, features):
    eij, sij, a_full, b_full = pl.pallas_call(
        _prep_kernel,
        out_shape=[
            jax.ShapeDtypeStruct((V, V), jnp.float32),
            jax.ShapeDtypeStruct((V, V), jnp.float32),
            jax.ShapeDtypeStruct((V, DD), jnp.float32),
            jax.ShapeDtypeStruct((V, DD), jnp.float32),
        ],
    )(costheta, features, _Pa, _Pb)

    c_flat = pl.pallas_call(
        _stream_kernel,
        grid=(V // BI,),
        in_specs=[
            pl.BlockSpec((1, BI, DD), lambda i: (i, 0, 0)),
            pl.BlockSpec((V, DD), lambda i: (0, 0)),
        ],
        out_specs=pl.BlockSpec(memory_space=pl.ANY),
        out_shape=jax.ShapeDtypeStruct((V, V, DD), jnp.float32),
        scratch_shapes=[
            pltpu.VMEM((NBUF, BI, V, DD), jnp.float32),
            pltpu.SemaphoreType.DMA((NBUF,)),
        ],
    )(a_full.reshape(V // BI, BI, DD), b_full)
    return (eij, sij, c_flat.reshape(V, V, D, D))
